# baseline (speedup 1.0000x reference)
import functools

import jax
import jax.numpy as jnp
from jax import lax
from jax.experimental import pallas as pl
from jax.experimental.pallas import tpu as pltpu
from jax.experimental.pallas import tpu_sc as plsc

NUM_NODES = 1000000
D = 64
B = 16384
NC = 2
NS = 16
L = 16
NW = NC * NS
BPW = B // NW
CHUNK = 128
NCH = BPW // CHUNK
RB = 4000


HN = NUM_NODES // 2


def _relayout_body(a1_ref, a2_ref, b1_ref, b2_ref, oa_ref, ob_ref):
    oa_ref[...] = jnp.concatenate([a1_ref[...], a2_ref[...]], axis=1)
    ob_ref[...] = jnp.concatenate([b1_ref[...], b2_ref[...]], axis=1)


@jax.jit
def _relayout(node_emb, context_emb):
    nblk = HN // RB
    return pl.pallas_call(
        _relayout_body,
        grid=(nblk,),
        in_specs=[
            pl.BlockSpec((RB, D), lambda i: (i, 0)),
            pl.BlockSpec((RB, D), lambda i: (i + HN // RB, 0)),
            pl.BlockSpec((RB, D), lambda i: (i, 0)),
            pl.BlockSpec((RB, D), lambda i: (i + HN // RB, 0)),
        ],
        out_specs=[
            pl.BlockSpec((RB, 2 * D), lambda i: (i, 0)),
            pl.BlockSpec((RB, 2 * D), lambda i: (i, 0)),
        ],
        out_shape=[
            jax.ShapeDtypeStruct((HN, 2 * D), jnp.float32),
            jax.ShapeDtypeStruct((HN, 2 * D), jnp.float32),
        ],
    )(node_emb, node_emb, context_emb, context_emb)


def _gather_body(I_hbm, J_hbm, node2_hbm, ctx2_hbm, out_hbm,
                 idx_i, idx_j, gidx_i, gidx_j, bufs_i, bufs_j, out_v,
                 sem_a, sem_b):
    wid = lax.axis_index("s") * NC + lax.axis_index("c")
    base = wid * BPW

    pltpu.sync_copy(I_hbm.at[wid], idx_i)
    pltpu.sync_copy(J_hbm.at[wid], idx_j)

    hn = jnp.full((L,), HN, jnp.int32)
    for c in range(NCH):
        for q in range(CHUNK // L):
            s = pl.ds(q * L, L)
            vi = idx_i[pl.ds(c * CHUNK + q * L, L)]
            vj = idx_j[pl.ds(c * CHUNK + q * L, L)]
            gidx_i[c, s] = jnp.where(vi < hn, vi, vi - hn)
            gidx_j[c, s] = jnp.where(vj < hn, vj, vj - hn)

    sems = [sem_a, sem_b]
    lanes = lax.iota(jnp.int32, L)
    hn0 = jnp.full((L,), HN, jnp.int32)
    zero = jnp.zeros((L,), jnp.int32)
    dd = jnp.full((L,), D, jnp.int32)

    def fire(c, par):
        return (
            pltpu.async_copy(node2_hbm.at[gidx_i.at[c]], bufs_i.at[par],
                             sems[par]),
            pltpu.async_copy(ctx2_hbm.at[gidx_j.at[c]], bufs_j.at[par],
                             sems[par]),
        )

    def compute(c, par):
        buf_i = bufs_i.at[par]
        buf_j = bufs_j.at[par]

        def group(g, _):
            o = c * CHUNK + g * L
            sub_i = jnp.where(idx_i[pl.ds(o, L)] < hn0, zero, dd)
            sub_j = jnp.where(idx_j[pl.ds(o, L)] < hn0, zero, dd)
            rows = g * L + lanes

            def dstep(d, acc):
                vi = plsc.load_gather(buf_i, [rows, sub_i + d])
                vj = plsc.load_gather(buf_j, [rows, sub_j + d])
                return acc + vi * vj

            acc = lax.fori_loop(0, D, dstep, jnp.zeros((L,), jnp.float32))
            out_v[pl.ds(o, L)] = acc
            return _

        lax.fori_loop(0, CHUNK // L, group, 0)

    pend = fire(0, 0)
    for c in range(NCH):
        par = c % 2
        if c + 1 < NCH:
            nxt = fire(c + 1, 1 - par)
        for cp in pend:
            cp.wait()
        compute(c, par)
        if c + 1 < NCH:
            pend = nxt

    pltpu.sync_copy(out_v, out_hbm.at[pl.ds(base, BPW)])


@jax.jit
def _line_second(I2, J2, node_emb, context_emb):
    node2, ctx2 = _relayout(node_emb, context_emb)
    kern = functools.partial(
        pl.kernel,
        out_type=jax.ShapeDtypeStruct((B,), jnp.float32),
        mesh=plsc.VectorSubcoreMesh(core_axis_name="c", subcore_axis_name="s"),
        compiler_params=pltpu.CompilerParams(needs_layout_passes=False),
        scratch_types=[
            pltpu.VMEM((BPW,), jnp.int32),
            pltpu.VMEM((BPW,), jnp.int32),
            pltpu.VMEM((NCH, CHUNK), jnp.int32),
            pltpu.VMEM((NCH, CHUNK), jnp.int32),
            pltpu.VMEM((2, CHUNK, 2 * D), jnp.float32),
            pltpu.VMEM((2, CHUNK, 2 * D), jnp.float32),
            pltpu.VMEM((BPW,), jnp.float32),
            pltpu.SemaphoreType.DMA,
            pltpu.SemaphoreType.DMA,
        ],
    )(_gather_body)
    return kern(I2, J2, node2, ctx2)


def kernel(I, J, node_emb, context_emb):
    I2 = I.astype(jnp.int32).reshape(NW, BPW)
    J2 = J.astype(jnp.int32).reshape(NW, BPW)
    return _line_second(I2, J2, node_emb, context_emb)


# speedup vs baseline: 1.3595x; 1.3595x over previous
import functools

import jax
import jax.numpy as jnp
from jax import lax
from jax.experimental import pallas as pl
from jax.experimental.pallas import tpu as pltpu
from jax.experimental.pallas import tpu_sc as plsc

NUM_NODES = 1000000
D = 64
B = 16384
NC = 2
NS = 16
L = 16
NW = NC * NS
BPW = B // NW
G = 8
NG = NUM_NODES // G
KT = 40
CHUNK = 128
NCH = BPW // CHUNK
NQ = BPW // L


def _pass(tbl_hbm, s_hbm, out_hbm, idx_v, rowbuf, staging, sem, wid):
    pltpu.sync_copy(s_hbm.at[wid], idx_v)
    my_lo = idx_v[pl.ds(0, L)][0]
    my_hi = idx_v[pl.ds(BPW - L, L)][15]
    g0 = lax.shift_right_logical(my_lo, 3)
    g1 = lax.shift_right_logical(my_hi, 3)
    nchunk = (g1 - g0) // KT + 1

    lanes = lax.iota(jnp.int32, L)

    def gbase(i):
        return jnp.minimum(g0 + i * KT, NG - KT)

    def fire(i, par):
        pltpu.async_copy(tbl_hbm.at[pl.ds(gbase(i), KT)],
                         staging.at[par], sem)

    def drain(par):
        pltpu.make_async_copy(tbl_hbm.at[pl.ds(0, KT)],
                              staging.at[par], sem).wait()

    fire(0, 0)

    def step(i, carry):
        par = i % 2

        @pl.when(i + 1 < nchunk)
        def _fire_next():
            fire(i + 1, 1 - par)

        drain(par)

        rlo = gbase(i) * G
        rhi = rlo + KT * G
        buf = staging.at[par]

        for q in range(NQ):
            v = idx_v[pl.ds(q * L, L)]
            lo_q = v[0]
            hi_q = v[15]

            @pl.when(jnp.logical_and(lo_q < rhi, hi_q >= rlo))
            def _pick():
                inside = jnp.logical_and(v >= rlo, v < rhi)
                lp = jnp.clip(v - rlo, 0, KT * G - 1)
                gq = lax.shift_right_logical(lp, 3)
                rq = lp & jnp.full((L,), 7, jnp.int32)
                rows = q * L + lanes
                r2 = lax.shift_right_logical(rows, 1)
                cb = (rows & jnp.full((L,), 1, jnp.int32)) * D

                def dstep(d, carry2):
                    col = jnp.zeros((L,), jnp.int32) + d
                    vals = plsc.load_gather(buf, [gq, rq, col], mask=inside)
                    plsc.store_scatter(rowbuf, [r2, cb + col], vals,
                                       mask=inside)
                    return carry2

                lax.fori_loop(0, D, dstep, 0)

        return carry

    lax.fori_loop(0, nchunk, step, 0)
    pltpu.sync_copy(rowbuf, out_hbm.at[pl.ds(wid * (BPW // 2), BPW // 2)])


def _conveyor_body(sI_hbm, sJ_hbm, node_hbm, ctx_hbm, outA_hbm, outB_hbm,
                   idx_v, rowbuf, staging, sem):
    c = lax.axis_index("c")
    s = lax.axis_index("s")
    wid = c * NS + s
    _pass(node_hbm, sI_hbm, outA_hbm, idx_v, rowbuf, staging, sem, wid)
    _pass(ctx_hbm, sJ_hbm, outB_hbm, idx_v, rowbuf, staging, sem, wid)


def _dot_body(pI_hbm, pJ_hbm, outA_hbm, outB_hbm, out_hbm,
              sub_i, sub_j, gidx_i, gidx_j, bufs_i, bufs_j, out_v,
              sem_a, sem_b):
    c = lax.axis_index("c")
    s = lax.axis_index("s")
    wid = c * NS + s
    base = wid * BPW

    pltpu.sync_copy(pI_hbm.at[wid], sub_i)
    pltpu.sync_copy(pJ_hbm.at[wid], sub_j)
    for ch in range(NCH):
        for q in range(CHUNK // L):
            sl = pl.ds(q * L, L)
            gidx_i[ch, sl] = lax.shift_right_logical(
                sub_i[pl.ds(ch * CHUNK + q * L, L)], 1)
            gidx_j[ch, sl] = lax.shift_right_logical(
                sub_j[pl.ds(ch * CHUNK + q * L, L)], 1)

    sems = [sem_a, sem_b]
    lanes = lax.iota(jnp.int32, L)
    one = jnp.full((L,), 1, jnp.int32)

    def fire(ch, par):
        return (
            pltpu.async_copy(outA_hbm.at[gidx_i.at[ch]], bufs_i.at[par],
                             sems[par]),
            pltpu.async_copy(outB_hbm.at[gidx_j.at[ch]], bufs_j.at[par],
                             sems[par]),
        )

    def compute(ch, par):
        buf_i = bufs_i.at[par]
        buf_j = bufs_j.at[par]

        def group(g, carry):
            o = ch * CHUNK + g * L
            hi_ = (sub_i[pl.ds(o, L)] & one) * D
            hj_ = (sub_j[pl.ds(o, L)] & one) * D
            rows = g * L + lanes

            def dstep(d, acc):
                vi = plsc.load_gather(buf_i, [rows, hi_ + d])
                vj = plsc.load_gather(buf_j, [rows, hj_ + d])
                return acc + vi * vj

            acc = lax.fori_loop(0, D, dstep, jnp.zeros((L,), jnp.float32))
            out_v[pl.ds(o, L)] = acc
            return carry

        lax.fori_loop(0, CHUNK // L, group, 0)

    pend = fire(0, 0)
    for ch in range(NCH):
        par = ch % 2
        if ch + 1 < NCH:
            nxt = fire(ch + 1, 1 - par)
        for cp in pend:
            cp.wait()
        compute(ch, par)
        if ch + 1 < NCH:
            pend = nxt

    pltpu.sync_copy(out_v, out_hbm.at[pl.ds(base, BPW)])


@jax.jit
def _line_second(I, J, node_emb, context_emb):
    I32 = I.astype(jnp.int32)
    J32 = J.astype(jnp.int32)
    permI = jnp.argsort(I32)
    permJ = jnp.argsort(J32)
    sI = I32[permI]
    sJ = J32[permJ]
    ar = jnp.arange(B, dtype=jnp.int32)
    invI = jnp.zeros((B,), jnp.int32).at[permI].set(ar)
    invJ = jnp.zeros((B,), jnp.int32).at[permJ].set(ar)
    sI2 = sI.reshape(NW, BPW)
    sJ2 = sJ.reshape(NW, BPW)
    pI2 = invI.reshape(NW, BPW)
    pJ2 = invJ.reshape(NW, BPW)
    node3 = node_emb.reshape(NG, G, D)
    ctx3 = context_emb.reshape(NG, G, D)

    conveyor = functools.partial(
        pl.kernel,
        out_type=(jax.ShapeDtypeStruct((B // 2, 2 * D), jnp.float32),
                  jax.ShapeDtypeStruct((B // 2, 2 * D), jnp.float32)),
        mesh=plsc.VectorSubcoreMesh(core_axis_name="c", subcore_axis_name="s"),
        compiler_params=pltpu.CompilerParams(needs_layout_passes=False),
        scratch_types=[
            pltpu.VMEM((BPW,), jnp.int32),
            pltpu.VMEM((BPW // 2, 2 * D), jnp.float32),
            pltpu.VMEM((2, KT, G, D), jnp.float32),
            pltpu.SemaphoreType.DMA,
        ],
    )(_conveyor_body)
    outA2, outB2 = conveyor(sI2, sJ2, node3, ctx3)

    dot = functools.partial(
        pl.kernel,
        out_type=jax.ShapeDtypeStruct((B,), jnp.float32),
        mesh=plsc.VectorSubcoreMesh(core_axis_name="c", subcore_axis_name="s"),
        compiler_params=pltpu.CompilerParams(needs_layout_passes=False),
        scratch_types=[
            pltpu.VMEM((BPW,), jnp.int32),
            pltpu.VMEM((BPW,), jnp.int32),
            pltpu.VMEM((NCH, CHUNK), jnp.int32),
            pltpu.VMEM((NCH, CHUNK), jnp.int32),
            pltpu.VMEM((2, CHUNK, 2 * D), jnp.float32),
            pltpu.VMEM((2, CHUNK, 2 * D), jnp.float32),
            pltpu.VMEM((BPW,), jnp.float32),
            pltpu.SemaphoreType.DMA,
            pltpu.SemaphoreType.DMA,
        ],
    )(_dot_body)
    return dot(pI2, pJ2, outA2, outB2)


def kernel(I, J, node_emb, context_emb):
    return _line_second(I, J, node_emb, context_emb)


# speedup vs baseline: 1.3620x; 1.0018x over previous
import functools

import jax
import jax.numpy as jnp
from jax import lax
from jax.experimental import pallas as pl
from jax.experimental.pallas import tpu as pltpu
from jax.experimental.pallas import tpu_sc as plsc

NUM_NODES = 1000000
D = 64
B = 16384
NC = 2
NS = 16
L = 16
NW = NC * NS
BPW = B // NW
G = 8
NG = NUM_NODES // G
KT = 40
CHUNK = 128
NCH = BPW // CHUNK
NQ = BPW // L


def _pass(tbl_hbm, s_hbm, out_hbm, idx_v, rowbuf, staging, sem, wid):
    pltpu.sync_copy(s_hbm.at[wid], idx_v)
    my_lo = idx_v[pl.ds(0, L)][0]
    my_hi = idx_v[pl.ds(BPW - L, L)][15]
    g0 = lax.shift_right_logical(my_lo, 3)
    g1 = lax.shift_right_logical(my_hi, 3)
    nchunk = (g1 - g0) // KT + 1

    lanes = lax.iota(jnp.int32, L)

    def gbase(i):
        return jnp.minimum(g0 + i * KT, NG - KT)

    def fire(i, par):
        pltpu.async_copy(tbl_hbm.at[pl.ds(gbase(i), KT)],
                         staging.at[par], sem)

    def drain(par):
        pltpu.make_async_copy(tbl_hbm.at[pl.ds(0, KT)],
                              staging.at[par], sem).wait()

    fire(0, 0)

    def step(i, carry):
        par = i % 2

        @pl.when(i + 1 < nchunk)
        def _fire_next():
            fire(i + 1, 1 - par)

        drain(par)

        rlo = gbase(i) * G
        rhi = rlo + KT * G
        buf = staging.at[par]

        for q in range(NQ):
            v = idx_v[pl.ds(q * L, L)]
            lo_q = v[0]
            hi_q = v[15]

            @pl.when(jnp.logical_and(lo_q < rhi, hi_q >= rlo))
            def _pick():
                inside = jnp.logical_and(v >= rlo, v < rhi)
                lp = jnp.clip(v - rlo, 0, KT * G - 1)
                gq = lax.shift_right_logical(lp, 3)
                rq = lp & jnp.full((L,), 7, jnp.int32)
                rows = q * L + lanes
                r2 = lax.shift_right_logical(rows, 1)
                cb = (rows & jnp.full((L,), 1, jnp.int32)) * D

                def dstep(d, carry2):
                    col = jnp.zeros((L,), jnp.int32) + d
                    vals = plsc.load_gather(buf, [gq, rq, col], mask=inside)
                    plsc.store_scatter(rowbuf, [r2, cb + col], vals,
                                       mask=inside)
                    return carry2

                lax.fori_loop(0, D, dstep, 0)

        return carry

    lax.fori_loop(0, nchunk, step, 0)
    pltpu.sync_copy(rowbuf, out_hbm.at[pl.ds(wid * (BPW // 2), BPW // 2)])


def _conveyor_body(sI_hbm, sJ_hbm, node_hbm, ctx_hbm, outA_hbm, outB_hbm,
                   idx_v, rowbuf, staging, sem):
    c = lax.axis_index("c")
    s = lax.axis_index("s")
    wid = c * NS + s
    _pass(node_hbm, sI_hbm, outA_hbm, idx_v, rowbuf, staging, sem, wid)
    _pass(ctx_hbm, sJ_hbm, outB_hbm, idx_v, rowbuf, staging, sem, wid)


def _dot_body(pI_hbm, pJ_hbm, outA_hbm, outB_hbm, out_hbm,
              sub_i, sub_j, gidx_i, gidx_j, bufs_i, bufs_j, out_v,
              sem_a, sem_b):
    c = lax.axis_index("c")
    s = lax.axis_index("s")
    wid = c * NS + s
    base = wid * BPW

    pltpu.sync_copy(pI_hbm.at[wid], sub_i)
    pltpu.sync_copy(pJ_hbm.at[wid], sub_j)
    for ch in range(NCH):
        for q in range(CHUNK // L):
            sl = pl.ds(q * L, L)
            gidx_i[ch, sl] = lax.shift_right_logical(
                sub_i[pl.ds(ch * CHUNK + q * L, L)], 1)
            gidx_j[ch, sl] = lax.shift_right_logical(
                sub_j[pl.ds(ch * CHUNK + q * L, L)], 1)

    sems = [sem_a, sem_b]
    lanes = lax.iota(jnp.int32, L)
    one = jnp.full((L,), 1, jnp.int32)

    def fire(ch, par):
        return (
            pltpu.async_copy(outA_hbm.at[gidx_i.at[ch]], bufs_i.at[par],
                             sems[par]),
            pltpu.async_copy(outB_hbm.at[gidx_j.at[ch]], bufs_j.at[par],
                             sems[par]),
        )

    def compute(ch, par):
        buf_i = bufs_i.at[par]
        buf_j = bufs_j.at[par]

        def group(g, carry):
            o = ch * CHUNK + g * L
            hi_ = (sub_i[pl.ds(o, L)] & one) * D
            hj_ = (sub_j[pl.ds(o, L)] & one) * D
            rows = g * L + lanes

            def dstep(d, acc):
                vi = plsc.load_gather(buf_i, [rows, hi_ + d])
                vj = plsc.load_gather(buf_j, [rows, hj_ + d])
                return acc + vi * vj

            acc = lax.fori_loop(0, D, dstep, jnp.zeros((L,), jnp.float32))
            out_v[pl.ds(o, L)] = acc
            return carry

        lax.fori_loop(0, CHUNK // L, group, 0)

    pend = fire(0, 0)
    for ch in range(NCH):
        par = ch % 2
        if ch + 1 < NCH:
            nxt = fire(ch + 1, 1 - par)
        for cp in pend:
            cp.wait()
        compute(ch, par)
        if ch + 1 < NCH:
            pend = nxt

    pltpu.sync_copy(out_v, out_hbm.at[pl.ds(base, BPW)])


@jax.jit
def _line_second(I, J, node3, ctx3):
    I32 = I.astype(jnp.int32)
    J32 = J.astype(jnp.int32)
    permI = jnp.argsort(I32)
    permJ = jnp.argsort(J32)
    sI = I32[permI]
    sJ = J32[permJ]
    ar = jnp.arange(B, dtype=jnp.int32)
    invI = jnp.zeros((B,), jnp.int32).at[permI].set(ar)
    invJ = jnp.zeros((B,), jnp.int32).at[permJ].set(ar)
    sI2 = sI.reshape(NW, BPW)
    sJ2 = sJ.reshape(NW, BPW)
    pI2 = invI.reshape(NW, BPW)
    pJ2 = invJ.reshape(NW, BPW)
    conveyor = functools.partial(
        pl.kernel,
        out_type=(jax.ShapeDtypeStruct((B // 2, 2 * D), jnp.float32),
                  jax.ShapeDtypeStruct((B // 2, 2 * D), jnp.float32)),
        mesh=plsc.VectorSubcoreMesh(core_axis_name="c", subcore_axis_name="s"),
        compiler_params=pltpu.CompilerParams(needs_layout_passes=False),
        scratch_types=[
            pltpu.VMEM((BPW,), jnp.int32),
            pltpu.VMEM((BPW // 2, 2 * D), jnp.float32),
            pltpu.VMEM((2, KT, G, D), jnp.float32),
            pltpu.SemaphoreType.DMA,
        ],
    )(_conveyor_body)
    outA2, outB2 = conveyor(sI2, sJ2, node3, ctx3)

    dot = functools.partial(
        pl.kernel,
        out_type=jax.ShapeDtypeStruct((B,), jnp.float32),
        mesh=plsc.VectorSubcoreMesh(core_axis_name="c", subcore_axis_name="s"),
        compiler_params=pltpu.CompilerParams(needs_layout_passes=False),
        scratch_types=[
            pltpu.VMEM((BPW,), jnp.int32),
            pltpu.VMEM((BPW,), jnp.int32),
            pltpu.VMEM((NCH, CHUNK), jnp.int32),
            pltpu.VMEM((NCH, CHUNK), jnp.int32),
            pltpu.VMEM((2, CHUNK, 2 * D), jnp.float32),
            pltpu.VMEM((2, CHUNK, 2 * D), jnp.float32),
            pltpu.VMEM((BPW,), jnp.float32),
            pltpu.SemaphoreType.DMA,
            pltpu.SemaphoreType.DMA,
        ],
    )(_dot_body)
    return dot(pI2, pJ2, outA2, outB2)


def kernel(I, J, node_emb, context_emb):
    node3 = node_emb.reshape(NG, G, D)
    ctx3 = context_emb.reshape(NG, G, D)
    return _line_second(I, J, node3, ctx3)


# speedup vs baseline: 2.3257x; 1.7075x over previous
import functools

import jax
import jax.numpy as jnp
from jax import lax
from jax.experimental import pallas as pl
from jax.experimental.pallas import tpu as pltpu
from jax.experimental.pallas import tpu_sc as plsc

NUM_NODES = 1000000
D = 64
B = 16384
NC = 2
NS = 16
L = 16
NW = NC * NS
BPW = B // NW
ST = L
NSTG = BPW // ST
G = 8


def _body(I_hbm, J_hbm, node_hbm, ctx_hbm, out_hbm,
          idx_i, idx_j, bufs_i, bufs_j, out_v, sem_a, sem_b):
    wid = lax.axis_index("s") * NC + lax.axis_index("c")
    base = wid * BPW

    pltpu.sync_copy(I_hbm.at[wid], idx_i)
    pltpu.sync_copy(J_hbm.at[wid], idx_j)

    sems = [sem_a, sem_b]
    lanes = lax.iota(jnp.int32, L)
    seven = jnp.full((L,), 7, jnp.int32)

    def fire(s, par):
        gi = lax.shift_right_logical(idx_i[pl.ds(s * L, L)], 3)
        gj = lax.shift_right_logical(idx_j[pl.ds(s * L, L)], 3)
        for k in range(L):
            pltpu.async_copy(node_hbm.at[gi[k]], bufs_i.at[par, k],
                             sems[par])
            pltpu.async_copy(ctx_hbm.at[gj[k]], bufs_j.at[par, k],
                             sems[par])

    def drain(par):
        pltpu.make_async_copy(node_hbm.at[pl.ds(0, ST)], bufs_i.at[par],
                              sems[par]).wait()
        pltpu.make_async_copy(ctx_hbm.at[pl.ds(0, ST)], bufs_j.at[par],
                              sems[par]).wait()

    def compute(s, par):
        sub_i = idx_i[pl.ds(s * L, L)] & seven
        sub_j = idx_j[pl.ds(s * L, L)] & seven
        buf_i = bufs_i.at[par]
        buf_j = bufs_j.at[par]

        def dstep(d, acc):
            col = jnp.zeros((L,), jnp.int32) + d
            vi = plsc.load_gather(buf_i, [lanes, sub_i, col])
            vj = plsc.load_gather(buf_j, [lanes, sub_j, col])
            return acc + vi * vj

        acc = lax.fori_loop(0, D, dstep, jnp.zeros((L,), jnp.float32))
        out_v[pl.ds(s * L, L)] = acc

    fire(0, 0)
    for s in range(NSTG):
        par = s % 2
        if s + 1 < NSTG:
            fire(s + 1, 1 - par)
        drain(par)
        compute(s, par)

    pltpu.sync_copy(out_v, out_hbm.at[pl.ds(base, BPW)])


@jax.jit
def _line_second(I2, J2, node3, ctx3):
    kern = functools.partial(
        pl.kernel,
        out_type=jax.ShapeDtypeStruct((B,), jnp.float32),
        mesh=plsc.VectorSubcoreMesh(core_axis_name="c", subcore_axis_name="s"),
        compiler_params=pltpu.CompilerParams(needs_layout_passes=False),
        scratch_types=[
            pltpu.VMEM((BPW,), jnp.int32),
            pltpu.VMEM((BPW,), jnp.int32),
            pltpu.VMEM((2, ST, G, D), jnp.float32),
            pltpu.VMEM((2, ST, G, D), jnp.float32),
            pltpu.VMEM((BPW,), jnp.float32),
            pltpu.SemaphoreType.DMA,
            pltpu.SemaphoreType.DMA,
        ],
    )(_body)
    return kern(I2, J2, node3, ctx3)


def kernel(I, J, node_emb, context_emb):
    I2 = I.astype(jnp.int32).reshape(NW, BPW)
    J2 = J.astype(jnp.int32).reshape(NW, BPW)
    node3 = node_emb.reshape(NUM_NODES // G, G, D)
    ctx3 = context_emb.reshape(NUM_NODES // G, G, D)
    return _line_second(I2, J2, node3, ctx3)
